# baseline (speedup 1.0000x reference)
import functools

import jax
import jax.numpy as jnp
from jax import lax
from jax.experimental import pallas as pl
from jax.experimental.pallas import tpu as pltpu
from jax.experimental.pallas import tpu_sc as plsc

N_BANK = 1_000_000
D_DIM = 128
B_TOK = 16384

_NC = 2
_NS = 16
_NW = _NC * _NS
_B_PER_W = B_TOK // _NW
_BLK = 128
_NBLK = _B_PER_W // _BLK
_LANES = 16

_mesh = plsc.VectorSubcoreMesh(core_axis_name="c", subcore_axis_name="s")


@functools.partial(
    pl.kernel,
    mesh=_mesh,
    out_type=jax.ShapeDtypeStruct((D_DIM // 8, B_TOK // 128, 8, 128), jnp.float32),
    compiler_params=pltpu.CompilerParams(needs_layout_passes=False),
    scratch_types=[
        pltpu.VMEM((_B_PER_W,), jnp.int32),
        pltpu.VMEM((_BLK, D_DIM), jnp.float32),
        pltpu.VMEM((_BLK, D_DIM), jnp.float32),
        pltpu.VMEM((D_DIM // 8, 8, _BLK), jnp.float32),
        pltpu.VMEM((D_DIM // 8, 8, _BLK), jnp.float32),
        pltpu.SemaphoreType.DMA,
        pltpu.SemaphoreType.DMA,
    ],
)
def _gather_t(idx_hbm, mbt_hbm, out_hbm, idx_v, rows0, rows1, tb0, tb1, gsem, osem):
    wid = lax.axis_index("s") * _NC + lax.axis_index("c")
    base = wid * _B_PER_W
    pltpu.sync_copy(idx_hbm.at[pl.ds(base, _B_PER_W)], idx_v)

    iota = lax.iota(jnp.int32, _LANES)

    rows_bufs = (rows0, rows1)
    tb_bufs = (tb0, tb1)

    def start_gather(j):
        return pltpu.async_copy(
            mbt_hbm.at[idx_v.at[pl.ds(j * _BLK, _BLK)]], rows_bufs[j % 2], gsem
        )

    def transpose_block(j):
        rows_j = rows_bufs[j % 2]
        tb_j = tb_bufs[j % 2]

        def dbody(d, _):
            dvec = jnp.full((_LANES,), 0, jnp.int32) + d
            tr = d // 8
            r = lax.rem(d, 8)
            for c16 in range(_BLK // _LANES):
                cvec = iota + (c16 * _LANES)
                vals = plsc.load_gather(rows_j, [cvec, dvec])
                tb_j[tr, r, pl.ds(c16 * _LANES, _LANES)] = vals
            return 0

        lax.fori_loop(0, D_DIM, dbody, 0, unroll=2)

    def start_store(j):
        jg = wid * _NBLK + j
        return pltpu.async_copy(tb_bufs[j % 2], out_hbm.at[:, jg], osem)

    g0 = start_gather(0)
    g1 = start_gather(1)
    g0.wait()
    transpose_block(0)
    s0 = start_store(0)
    g1.wait()
    g2 = start_gather(2)
    transpose_block(1)
    s1 = start_store(1)
    g2.wait()
    g3 = start_gather(3)
    s0.wait()
    transpose_block(2)
    s2 = start_store(2)
    g3.wait()
    s1.wait()
    transpose_block(3)
    s3 = start_store(3)
    s2.wait()
    s3.wait()


def kernel(index, membank):
    mbt = membank.T
    out4 = _gather_t(index, mbt)
    return out4.transpose(0, 2, 1, 3).reshape(D_DIM, B_TOK)


# speedup vs baseline: 1.4808x; 1.4808x over previous
import functools

import jax
import jax.numpy as jnp
from jax import lax
from jax.experimental import pallas as pl
from jax.experimental.pallas import tpu as pltpu
from jax.experimental.pallas import tpu_sc as plsc

N_BANK = 1_000_000
D_DIM = 128
B_TOK = 16384

_NC = 2
_NS = 16
_NW = _NC * _NS
_B_PER_W = B_TOK // _NW
_BLK = 128
_NBLK = _B_PER_W // _BLK
_LANES = 16

_mesh = plsc.VectorSubcoreMesh(core_axis_name="c", subcore_axis_name="s")


@functools.partial(
    pl.kernel,
    mesh=_mesh,
    out_type=jax.ShapeDtypeStruct((D_DIM // 8, B_TOK // 128, 8, 128), jnp.float32),
    compiler_params=pltpu.CompilerParams(needs_layout_passes=False),
    scratch_types=[
        pltpu.VMEM((_B_PER_W,), jnp.int32),
        pltpu.VMEM((_BLK, D_DIM), jnp.float32),
        pltpu.VMEM((_BLK, D_DIM), jnp.float32),
        pltpu.VMEM((D_DIM // 8, 8, _BLK), jnp.float32),
        pltpu.VMEM((D_DIM // 8, 8, _BLK), jnp.float32),
        pltpu.SemaphoreType.DMA,
        pltpu.SemaphoreType.DMA,
    ],
)
def _gather_t(idx_hbm, mbt_hbm, out_hbm, idx_v, rows0, rows1, tb0, tb1, gsem, osem):
    wid = lax.axis_index("s") * _NC + lax.axis_index("c")
    base = wid * _B_PER_W
    pltpu.sync_copy(idx_hbm.at[pl.ds(base, _B_PER_W)], idx_v)

    iota = lax.iota(jnp.int32, _LANES)

    rows_bufs = (rows0, rows1)
    tb_bufs = (tb0, tb1)

    def start_gather(j):
        return pltpu.async_copy(
            mbt_hbm.at[idx_v.at[pl.ds(j * _BLK, _BLK)]], rows_bufs[j % 2], gsem
        )

    cvecs = tuple(iota + (c16 * _LANES) for c16 in range(_BLK // _LANES))

    def transpose_block(j):
        rows_j = rows_bufs[j % 2]
        tb_j = tb_bufs[j % 2]

        @plsc.parallel_loop(0, D_DIM, unroll=4)
        def dbody(d):
            dvec = jnp.full((_LANES,), 0, jnp.int32) + d
            tr = lax.shift_right_logical(d, 3)
            r = lax.bitwise_and(d, 7)
            for c16 in range(_BLK // _LANES):
                vals = plsc.load_gather(rows_j, [cvecs[c16], dvec])
                tb_j[tr, r, pl.ds(c16 * _LANES, _LANES)] = vals

    def start_store(j):
        jg = wid * _NBLK + j
        return pltpu.async_copy(tb_bufs[j % 2], out_hbm.at[:, jg], osem)

    g0 = start_gather(0)
    g1 = start_gather(1)
    g0.wait()
    transpose_block(0)
    s0 = start_store(0)
    g1.wait()
    g2 = start_gather(2)
    transpose_block(1)
    s1 = start_store(1)
    g2.wait()
    g3 = start_gather(3)
    s0.wait()
    transpose_block(2)
    s2 = start_store(2)
    g3.wait()
    s1.wait()
    transpose_block(3)
    s3 = start_store(3)
    s2.wait()
    s3.wait()


def kernel(index, membank):
    mbt = membank.T
    out4 = _gather_t(index, mbt)
    return out4.transpose(0, 2, 1, 3).reshape(D_DIM, B_TOK)
